# baseline (speedup 1.0000x reference)
import jax
import jax.numpy as jnp
from jax import lax
from jax.experimental import pallas as pl
from jax.experimental.pallas import tpu as pltpu
from jax.experimental.pallas import tpu_sc as plsc

N = 10000
E = 160000
D_IN = 128
H = 128
G = 64

NC = 2
NS = 16
NW = NC * NS
CHUNK = 128
CPW = 40
E_PAD = NW * CPW * CHUNK
NEXTRA = 0
N_ACC = 10240
RPS = N_ACC // NS
ZC = ((RPS + 15) // 16) * 16
ROWS_BLK = 1000
N_BLKS = N // ROWS_BLK


def _in_mlp_body(x_ref, w_ref, b_ref, o_ref):
    y = jnp.dot(x_ref[...], w_ref[...], preferred_element_type=jnp.float32)
    o_ref[...] = jnp.maximum(y + b_ref[...], 0.0)


def _in_mlp(x, W, b2d):
    return pl.pallas_call(
        _in_mlp_body,
        grid=(N_BLKS,),
        in_specs=[
            pl.BlockSpec((ROWS_BLK, D_IN), lambda i: (i, 0)),
            pl.BlockSpec((D_IN, H), lambda i: (0, 0)),
            pl.BlockSpec((1, H), lambda i: (0, 0)),
        ],
        out_specs=pl.BlockSpec((ROWS_BLK, H), lambda i: (i, 0)),
        out_shape=jax.ShapeDtypeStruct((N, H), jnp.float32),
    )(x, W, b2d)


def _edge_agg_body(h_hbm, e_hbm, acc_out, cnt_out,
                   esrc, edst, msg, ones_v, zcnt, acc, cnt,
                   semg0, semg1, sems, semc):
    c = lax.axis_index("c")
    s = lax.axis_index("s")
    wid = c * NS + s
    base = wid * CPW

    pltpu.sync_copy(e_hbm.at[0, pl.ds(base, CPW)], esrc)
    pltpu.sync_copy(e_hbm.at[1, pl.ds(base, CPW)], edst)

    zeros16 = jnp.zeros((16,), jnp.float32)
    ones16 = jnp.ones((16,), jnp.float32)

    def zrow(r, carry):
        for k in range(H // 16):
            msg[0, r, pl.ds(k * 16, 16)] = zeros16
        return carry
    lax.fori_loop(0, CHUNK, zrow, 0)

    def zc(i, carry):
        zcnt[pl.ds(i * 16, 16)] = zeros16
        return carry
    lax.fori_loop(0, ZC // 16, zc, 0)

    for k in range(CHUNK // 16):
        ones_v[pl.ds(k * 16, 16)] = ones16

    for k in range(RPS // CHUNK):
        pltpu.sync_copy(msg.at[0], acc.at[pl.ds(s * RPS + k * CHUNK, CHUNK)])
    rem = RPS - (RPS // CHUNK) * CHUNK
    if rem:
        pltpu.sync_copy(msg.at[0, pl.ds(0, rem)],
                        acc.at[pl.ds(s * RPS + (RPS // CHUNK) * CHUNK, rem)])
    pltpu.sync_copy(zcnt.at[pl.ds(0, RPS)], cnt.at[pl.ds(s * RPS, RPS)])

    plsc.subcore_barrier()

    pltpu.async_copy(h_hbm.at[esrc.at[0]], msg.at[0], semg0)
    pltpu.async_copy(h_hbm.at[esrc.at[1]], msg.at[1], semg1)

    def do_chunk(j, b):
        semg = semg0 if b == 0 else semg1
        pltpu.make_async_copy(h_hbm.at[esrc.at[j]], msg.at[b], semg).wait()
        d = pltpu.async_copy(msg.at[b], acc.at[edst.at[j]], sems, add=True)

        @pl.when(j > 0)
        def _():
            pltpu.make_async_copy(ones_v, cnt.at[edst.at[j]], semc).wait()
        pltpu.async_copy(ones_v, cnt.at[edst.at[j]], semc, add=True)
        d.wait()

        @pl.when(j + 2 < CPW)
        def _():
            pltpu.async_copy(h_hbm.at[esrc.at[j + 2]], msg.at[b], semg)

    def step(jj, carry):
        for b in range(2):
            do_chunk(jj * 2 + b, b)
        return carry
    lax.fori_loop(0, CPW // 2, step, 0)
    if CPW % 2:
        do_chunk(jnp.int32(CPW - 1), (CPW - 1) % 2)
    pltpu.make_async_copy(ones_v, cnt.at[edst.at[0]], semc).wait()

    plsc.subcore_barrier()

    pltpu.sync_copy(acc.at[pl.ds(s * RPS, RPS)],
                    acc_out.at[c, pl.ds(s * RPS, RPS)])
    pltpu.sync_copy(cnt.at[pl.ds(s * RPS, RPS)],
                    cnt_out.at[c, pl.ds(s * RPS, RPS)])


def _edge_agg(h, e_pad):
    mesh = plsc.VectorSubcoreMesh(core_axis_name="c", subcore_axis_name="s")
    f = pl.kernel(
        _edge_agg_body,
        out_type=[
            jax.ShapeDtypeStruct((NC, N_ACC, H), jnp.float32),
            jax.ShapeDtypeStruct((NC, N_ACC), jnp.float32),
        ],
        mesh=mesh,
        scratch_types=[
            pltpu.VMEM((CPW, CHUNK), jnp.int32),
            pltpu.VMEM((CPW, CHUNK), jnp.int32),
            pltpu.VMEM((2, CHUNK, H), jnp.float32),
            pltpu.VMEM((CHUNK,), jnp.float32),
            pltpu.VMEM((ZC,), jnp.float32),
            pltpu.VMEM_SHARED((N_ACC, H), jnp.float32),
            pltpu.VMEM_SHARED((N_ACC,), jnp.float32),
            pltpu.SemaphoreType.DMA,
            pltpu.SemaphoreType.DMA,
            pltpu.SemaphoreType.DMA,
            pltpu.SemaphoreType.DMA,
        ],
    )
    return f(h, e_pad)


def _tail_body(h_ref, a0_ref, a1_ref, c_ref, b_ref,
               wg_ref, bg_ref, w1_ref, b1_ref, w2_ref, b2_ref, bp_ref,
               out_ref, gsum, gcnt):
    i = pl.program_id(0)

    @pl.when(i == 0)
    def _():
        gsum[...] = jnp.zeros_like(gsum)
        gcnt[...] = jnp.zeros_like(gcnt)

    cnt = lax.dot_general(c_ref[0], jnp.ones((2, 1), jnp.float32),
                          (((0,), (0,)), ((), ())),
                          preferred_element_type=jnp.float32)
    bcol = lax.dot_general(b_ref[0].astype(jnp.float32),
                           jnp.ones((1, 1), jnp.float32),
                           (((0,), (0,)), ((), ())),
                           preferred_element_type=jnp.float32)
    agg = (a0_ref[0] + a1_ref[0]) / jnp.maximum(cnt, 1.0)
    h2 = jnp.dot(h_ref[...] + agg, wg_ref[...],
                 preferred_element_type=jnp.float32) + bg_ref[...]
    h2 = jnp.maximum(h2, 0.0)

    onehot = (bcol ==
              lax.broadcasted_iota(jnp.int32, (ROWS_BLK, G), 1)
              .astype(jnp.float32)).astype(jnp.float32)
    gsum[...] += lax.dot_general(onehot, h2, (((0,), (0,)), ((), ())),
                                 preferred_element_type=jnp.float32,
                                 precision=lax.Precision.HIGHEST)
    gcnt[...] += lax.dot_general(onehot, jnp.ones((ROWS_BLK, 1), jnp.float32),
                                 (((0,), (0,)), ((), ())),
                                 preferred_element_type=jnp.float32)

    @pl.when(i == N_BLKS - 1)
    def _():
        gmean = gsum[...] / jnp.maximum(gcnt[...], 1.0)
        hid = jnp.dot(gmean, w1_ref[...],
                      preferred_element_type=jnp.float32) + b1_ref[...]
        hid = jnp.maximum(hid, 0.0)
        props = jnp.dot(hid, w2_ref[...],
                        preferred_element_type=jnp.float32)
        props = props + b2_ref[...] + bp_ref[...]
        ii = lax.broadcasted_iota(jnp.int32, (G, 4), 1)
        props = jnp.where(ii == 1, jax.nn.sigmoid(props), props)
        props = jnp.where(ii == 2, jnp.maximum(props, 0.0) + 1.0, props)
        out_ref[...] = props


def _tail(h, acc, cpair, brow, W_gnn, bg2d, W_h1, b12d, W_h2, b22d, bp2d):
    full = lambda r, c_: pl.BlockSpec((r, c_), lambda i: (0, 0))
    rows = lambda c_: pl.BlockSpec((ROWS_BLK, c_), lambda i: (i, 0))
    return pl.pallas_call(
        _tail_body,
        grid=(N_BLKS,),
        in_specs=[
            rows(H),
            pl.BlockSpec((1, ROWS_BLK, H), lambda i: (0, i, 0)),
            pl.BlockSpec((1, ROWS_BLK, H), lambda i: (1, i, 0)),
            pl.BlockSpec((1, 2, ROWS_BLK), lambda i: (i, 0, 0)),
            pl.BlockSpec((1, 1, ROWS_BLK), lambda i: (i, 0, 0)),
            full(H, H), full(1, H), full(H, H // 2), full(1, H // 2),
            full(H // 2, 4), full(1, 4), full(1, 4),
        ],
        out_specs=pl.BlockSpec((G, 4), lambda i: (0, 0)),
        out_shape=jax.ShapeDtypeStruct((G, 4), jnp.float32),
        scratch_shapes=[
            pltpu.VMEM((G, H), jnp.float32),
            pltpu.VMEM((G, 1), jnp.float32),
        ],
        compiler_params=pltpu.CompilerParams(
            dimension_semantics=("arbitrary",)),
    )(h, acc, acc, cpair, brow, W_gnn, bg2d, W_h1, b12d, W_h2, b22d, bp2d)


def kernel(x, edge_index, batch, W_in, b_in, W_gnn, b_gnn,
           W_h1, b_h1, W_h2, b_h2, bias_props):
    pad = E_PAD - E
    pad_src = jnp.arange(pad, dtype=jnp.int32) % N
    pad_dst = N + (jnp.arange(pad, dtype=jnp.int32) % (N_ACC - N))
    e_pad = jnp.concatenate(
        [edge_index, jnp.stack([pad_src, pad_dst])],
        axis=1).reshape(2, E_PAD // CHUNK, CHUNK)

    h = _in_mlp(x, W_in, b_in.reshape(1, H))
    acc, cnt = _edge_agg(h, e_pad)

    out4 = _tail(
        h, acc,
        cnt[:, :N].reshape(2, N_BLKS, ROWS_BLK).transpose(1, 0, 2),
        batch.reshape(N_BLKS, 1, ROWS_BLK),
        W_gnn, b_gnn.reshape(1, H),
        W_h1, b_h1.reshape(1, H // 2),
        W_h2, b_h2.reshape(1, 4), bias_props.reshape(1, 4),
    )
    return (out4[:, 0], out4[:, 1], out4[:, 2], out4[:, 3])


# speedup vs baseline: 1.0501x; 1.0501x over previous
import jax
import jax.numpy as jnp
from jax import lax
from jax.experimental import pallas as pl
from jax.experimental.pallas import tpu as pltpu
from jax.experimental.pallas import tpu_sc as plsc

N = 10000
E = 160000
D_IN = 128
H = 128
G = 64

NC = 2
NS = 16
NW = NC * NS
CHUNK = 128
CPW = 40
E_PAD = NW * CPW * CHUNK
N_ACC = 10240
RPS = N_ACC // NS
ZC = ((RPS + 15) // 16) * 16
ROWS_BLK = 1000
N_BLKS = N // ROWS_BLK


def _in_mlp_body(x_ref, w_ref, b_ref, o_ref):
    y = jnp.dot(x_ref[...], w_ref[...], preferred_element_type=jnp.float32)
    o_ref[...] = jnp.maximum(y + b_ref[...], 0.0)


def _in_mlp(x, W, b2d):
    return pl.pallas_call(
        _in_mlp_body,
        grid=(N_BLKS,),
        in_specs=[
            pl.BlockSpec((ROWS_BLK, D_IN), lambda i: (i, 0)),
            pl.BlockSpec((D_IN, H), lambda i: (0, 0)),
            pl.BlockSpec((1, H), lambda i: (0, 0)),
        ],
        out_specs=pl.BlockSpec((ROWS_BLK, H), lambda i: (i, 0)),
        out_shape=jax.ShapeDtypeStruct((N, H), jnp.float32),
    )(x, W, b2d)


def _edge_agg_body(h_hbm, e_hbm, acc_out, cnt_out,
                   esrc, edst, msg, ones_v, zcnt, acc, cnt,
                   semg0, semg1, sems, semc):
    c = lax.axis_index("c")
    s = lax.axis_index("s")
    wid = c * NS + s
    base = wid * CPW

    pltpu.sync_copy(e_hbm.at[0, pl.ds(base, CPW)], esrc)
    pltpu.sync_copy(e_hbm.at[1, pl.ds(base, CPW)], edst)

    zeros16 = jnp.zeros((16,), jnp.float32)
    ones16 = jnp.ones((16,), jnp.float32)

    def zrow(r, carry):
        for k in range(H // 16):
            msg[0, r, pl.ds(k * 16, 16)] = zeros16
        return carry
    lax.fori_loop(0, CHUNK, zrow, 0)

    def zc(i, carry):
        zcnt[pl.ds(i * 16, 16)] = zeros16
        return carry
    lax.fori_loop(0, ZC // 16, zc, 0)

    for k in range(CHUNK // 16):
        ones_v[pl.ds(k * 16, 16)] = ones16

    for k in range(RPS // CHUNK):
        pltpu.sync_copy(msg.at[0], acc.at[pl.ds(s * RPS + k * CHUNK, CHUNK)])
    rem = RPS - (RPS // CHUNK) * CHUNK
    if rem:
        pltpu.sync_copy(msg.at[0, pl.ds(0, rem)],
                        acc.at[pl.ds(s * RPS + (RPS // CHUNK) * CHUNK, rem)])
    pltpu.sync_copy(zcnt.at[pl.ds(0, RPS)], cnt.at[pl.ds(s * RPS, RPS)])

    plsc.subcore_barrier()

    pltpu.async_copy(h_hbm.at[esrc.at[0]], msg.at[0], semg0)
    pltpu.async_copy(h_hbm.at[esrc.at[1]], msg.at[1], semg1)

    def do_chunk(j, b):
        semg = semg0 if b == 0 else semg1
        pltpu.make_async_copy(h_hbm.at[esrc.at[j]], msg.at[b], semg).wait()
        d = pltpu.async_copy(msg.at[b], acc.at[edst.at[j]], sems, add=True)

        @pl.when(j > 0)
        def _():
            pltpu.make_async_copy(ones_v, cnt.at[edst.at[j]], semc).wait()
        pltpu.async_copy(ones_v, cnt.at[edst.at[j]], semc, add=True)
        d.wait()

        @pl.when(j + 2 < CPW)
        def _():
            pltpu.async_copy(h_hbm.at[esrc.at[j + 2]], msg.at[b], semg)

    def step(jj, carry):
        for b in range(2):
            do_chunk(jj * 2 + b, b)
        return carry
    lax.fori_loop(0, CPW // 2, step, 0)
    if CPW % 2:
        do_chunk(jnp.int32(CPW - 1), (CPW - 1) % 2)
    pltpu.make_async_copy(ones_v, cnt.at[edst.at[0]], semc).wait()

    plsc.subcore_barrier()

    pltpu.sync_copy(acc.at[pl.ds(s * RPS, RPS)],
                    acc_out.at[c, pl.ds(s * RPS, RPS)])
    pltpu.sync_copy(cnt.at[pl.ds(s * RPS, RPS)],
                    cnt_out.at[c, pl.ds(s * RPS, RPS)])


def _edge_agg(h, e_pad):
    mesh = plsc.VectorSubcoreMesh(core_axis_name="c", subcore_axis_name="s")
    f = pl.kernel(
        _edge_agg_body,
        out_type=[
            jax.ShapeDtypeStruct((NC, N_ACC, H), jnp.float32),
            jax.ShapeDtypeStruct((NC, N_ACC), jnp.float32),
        ],
        mesh=mesh,
        scratch_types=[
            pltpu.VMEM((CPW, CHUNK), jnp.int32),
            pltpu.VMEM((CPW, CHUNK), jnp.int32),
            pltpu.VMEM((2, CHUNK, H), jnp.float32),
            pltpu.VMEM((CHUNK,), jnp.float32),
            pltpu.VMEM((ZC,), jnp.float32),
            pltpu.VMEM_SHARED((N_ACC, H), jnp.float32),
            pltpu.VMEM_SHARED((N_ACC,), jnp.float32),
            pltpu.SemaphoreType.DMA,
            pltpu.SemaphoreType.DMA,
            pltpu.SemaphoreType.DMA,
            pltpu.SemaphoreType.DMA,
        ],
    )
    return f(h, e_pad)


def _tail_body(h_ref, a0_ref, a1_ref, c_ref, b_ref,
               wg_ref, bg_ref, w1_ref, b1_ref, w2_ref, b2_ref, bp_ref,
               out_ref, gsum, gcnt):
    i = pl.program_id(0)

    @pl.when(i == 0)
    def _():
        gsum[...] = jnp.zeros_like(gsum)
        gcnt[...] = jnp.zeros_like(gcnt)

    cnt = lax.dot_general(c_ref[0], jnp.ones((2, 1), jnp.float32),
                          (((0,), (0,)), ((), ())),
                          preferred_element_type=jnp.float32)
    bcol = lax.dot_general(b_ref[0].astype(jnp.float32),
                           jnp.ones((1, 1), jnp.float32),
                           (((0,), (0,)), ((), ())),
                           preferred_element_type=jnp.float32)
    agg = (a0_ref[0] + a1_ref[0]) / jnp.maximum(cnt, 1.0)
    h2 = jnp.dot(h_ref[...] + agg, wg_ref[...],
                 preferred_element_type=jnp.float32) + bg_ref[...]
    h2 = jnp.maximum(h2, 0.0)

    onehot = (bcol ==
              lax.broadcasted_iota(jnp.int32, (ROWS_BLK, G), 1)
              .astype(jnp.float32)).astype(jnp.float32)
    gsum[...] += lax.dot_general(onehot, h2, (((0,), (0,)), ((), ())),
                                 preferred_element_type=jnp.float32,
                                 precision=lax.Precision.HIGHEST)
    gcnt[...] += lax.dot_general(onehot, jnp.ones((ROWS_BLK, 1), jnp.float32),
                                 (((0,), (0,)), ((), ())),
                                 preferred_element_type=jnp.float32)

    @pl.when(i == N_BLKS - 1)
    def _():
        gmean = gsum[...] / jnp.maximum(gcnt[...], 1.0)
        hid = jnp.dot(gmean, w1_ref[...],
                      preferred_element_type=jnp.float32) + b1_ref[...]
        hid = jnp.maximum(hid, 0.0)
        props = jnp.dot(hid, w2_ref[...],
                        preferred_element_type=jnp.float32)
        props = props + b2_ref[...] + bp_ref[...]
        ii = lax.broadcasted_iota(jnp.int32, (G, 4), 1)
        props = jnp.where(ii == 1, jax.nn.sigmoid(props), props)
        props = jnp.where(ii == 2, jnp.maximum(props, 0.0) + 1.0, props)
        out_ref[...] = props


def _tail(h, acc, cpair, brow, W_gnn, bg2d, W_h1, b12d, W_h2, b22d, bp2d):
    full = lambda r, c_: pl.BlockSpec((r, c_), lambda i: (0, 0))
    rows = lambda c_: pl.BlockSpec((ROWS_BLK, c_), lambda i: (i, 0))
    return pl.pallas_call(
        _tail_body,
        grid=(N_BLKS,),
        in_specs=[
            rows(H),
            pl.BlockSpec((1, ROWS_BLK, H), lambda i: (0, i, 0)),
            pl.BlockSpec((1, ROWS_BLK, H), lambda i: (1, i, 0)),
            pl.BlockSpec((1, 2, ROWS_BLK), lambda i: (i, 0, 0)),
            pl.BlockSpec((1, 1, ROWS_BLK), lambda i: (i, 0, 0)),
            full(H, H), full(1, H), full(H, H // 2), full(1, H // 2),
            full(H // 2, 4), full(1, 4), full(1, 4),
        ],
        out_specs=pl.BlockSpec((G, 4), lambda i: (0, 0)),
        out_shape=jax.ShapeDtypeStruct((G, 4), jnp.float32),
        scratch_shapes=[
            pltpu.VMEM((G, H), jnp.float32),
            pltpu.VMEM((G, 1), jnp.float32),
        ],
        compiler_params=pltpu.CompilerParams(
            dimension_semantics=("arbitrary",)),
    )(h, acc, acc, cpair, brow, W_gnn, bg2d, W_h1, b12d, W_h2, b22d, bp2d)


def kernel(x, edge_index, batch, W_in, b_in, W_gnn, b_gnn,
           W_h1, b_h1, W_h2, b_h2, bias_props):
    pad = E_PAD - E
    pad_src = jnp.arange(pad, dtype=jnp.int32) % N
    pad_dst = N + (jnp.arange(pad, dtype=jnp.int32) % (N_ACC - N))
    e_pad = jnp.concatenate(
        [edge_index, jnp.stack([pad_src, pad_dst])],
        axis=1).reshape(2, E_PAD // CHUNK, CHUNK)

    h = _in_mlp(x, W_in, b_in.reshape(1, H))
    acc, cnt = _edge_agg(h, e_pad)

    out4 = _tail(
        h, acc,
        cnt[:, :N].reshape(2, N_BLKS, ROWS_BLK).transpose(1, 0, 2),
        batch.reshape(N_BLKS, 1, ROWS_BLK),
        W_gnn, b_gnn.reshape(1, H),
        W_h1, b_h1.reshape(1, H // 2),
        W_h2, b_h2.reshape(1, 4), bias_props.reshape(1, 4),
    )
    return (out4[:, 0], out4[:, 1], out4[:, 2], out4[:, 3])
